# baseline (speedup 1.0000x reference)
import functools

import jax
import jax.numpy as jnp
from jax import lax
from jax.experimental import pallas as pl
from jax.experimental.pallas import tpu as pltpu
from jax.experimental.pallas import tpu_sc as plsc

N = 10000
E = 160000
D = 256
NG = 16
NCLS = 10
BN_EPS = 1e-5

H = D // 2
NT = 16
CH = 128
NCH = 80
EPT = NCH * CH
EPAD = NT * EPT
G = 16
NGR = NCH // G
RPT = 640
NP = NT * RPT
RB = 2000
GRID = N // RB


def _sc_agg_body(src_hbm, srcn_hbm, dst_hbm, hlr_hbm, outl_hbm, outr_hbm,
                 sidx_a, didx_a, rows0, rows1,
                 acc_sh, gsem0, gsem1, ssem0, ssem1):
    c = lax.axis_index("c")
    s = lax.axis_index("s")
    rows = (rows0, rows1)
    gsem = (gsem0, gsem1)
    ssem = (ssem0, ssem1)

    zvec = jnp.zeros((16,), jnp.float32)

    def zrow(i, carry):
        for k in range(8):
            rows0[i, pl.ds(k * 16, 16)] = zvec
        return carry

    lax.fori_loop(0, 64, zrow, 0)
    base = pl.multiple_of(s * RPT, RPT)
    for t in range(RPT // 64):
        pltpu.sync_copy(rows0.at[pl.ds(0, 64)], acc_sh.at[pl.ds(base + t * 64, 64)])
    plsc.subcore_barrier()

    def load_idx(sidx, didx, off):
        @pl.when(c == 0)
        def _():
            pltpu.sync_copy(src_hbm.at[s, pl.ds(off, G)], sidx)

        @pl.when(c == 1)
        def _():
            pltpu.sync_copy(srcn_hbm.at[s, pl.ds(off, G)], sidx)

        pltpu.sync_copy(dst_hbm.at[s, pl.ds(off, G)], didx)

    def gather(idx_ref, j, buf):
        return pltpu.async_copy(hlr_hbm.at[idx_ref.at[j]], rows[buf], gsem[buf])

    def group(k, carry):
        load_idx(sidx_a, didx_a, pl.multiple_of(k * G, G))
        pendg = gather(sidx_a, 0, 0)
        pends = [None, None]
        for t in range(G):
            cur = pendg
            if t + 1 < G:
                if pends[(t + 1) % 2] is not None:
                    pends[(t + 1) % 2].wait()
                pendg = gather(sidx_a, t + 1, (t + 1) % 2)
            cur.wait()
            pends[t % 2] = pltpu.async_copy(
                rows[t % 2], acc_sh.at[didx_a.at[t]], ssem[t % 2], add=True)
        pends[0].wait()
        pends[1].wait()
        return carry

    lax.fori_loop(0, NGR, group, 0)
    plsc.subcore_barrier()

    base_w = pl.multiple_of(jnp.minimum(base, N - RPT), 8)

    @pl.when(c == 0)
    def _():
        pltpu.sync_copy(acc_sh.at[pl.ds(base_w, RPT)], outl_hbm.at[pl.ds(base_w, RPT)])

    @pl.when(c == 1)
    def _():
        pltpu.sync_copy(acc_sh.at[pl.ds(base_w, RPT)], outr_hbm.at[pl.ds(base_w, RPT)])


_sc_agg = pl.kernel(
    _sc_agg_body,
    out_type=(
        jax.ShapeDtypeStruct((N, H), jnp.float32),
        jax.ShapeDtypeStruct((N, H), jnp.float32),
    ),
    mesh=plsc.VectorSubcoreMesh(core_axis_name="c", subcore_axis_name="s"),
    scratch_types=(
        pltpu.VMEM((G, CH), jnp.int32),
        pltpu.VMEM((G, CH), jnp.int32),
        pltpu.VMEM((CH, H), jnp.float32),
        pltpu.VMEM((CH, H), jnp.float32),
        pltpu.VMEM_SHARED((NP, H), jnp.float32),
        pltpu.SemaphoreType.DMA,
        pltpu.SemaphoreType.DMA,
        pltpu.SemaphoreType.DMA,
        pltpu.SemaphoreType.DMA,
    ),
)


def _lrelu(x):
    return jnp.where(x > 0, x, 0.01 * x)


def _mlp(se_ref, h_ref, al_ref, ar_ref, w1_ref, b1_ref, w2_ref, b2_ref):
    h = jnp.concatenate([h_ref[0], h_ref[1]], axis=1)
    a = jnp.concatenate([al_ref[...], ar_ref[...]], axis=1)
    z = se_ref[0, 0] * h + a
    z = jnp.dot(z, w1_ref[...], preferred_element_type=jnp.float32) + b1_ref[...]
    z = _lrelu(z)
    z = jnp.dot(z, w2_ref[...], preferred_element_type=jnp.float32) + b2_ref[...]
    return _lrelu(z)


def _mlp_split_body(se_ref, h_ref, al_ref, ar_ref,
                    w1_ref, b1_ref, w2_ref, b2_ref, out_ref):
    z = _mlp(se_ref, h_ref, al_ref, ar_ref, w1_ref, b1_ref, w2_ref, b2_ref)
    out_ref[0] = z[:, :H]
    out_ref[1] = z[:, H:]


_MLP_IN_SPECS = [
    pl.BlockSpec(memory_space=pltpu.SMEM),
    pl.BlockSpec((2, RB, H), lambda i: (0, i, 0)),
    pl.BlockSpec((RB, H), lambda i: (i, 0)),
    pl.BlockSpec((RB, H), lambda i: (i, 0)),
    pl.BlockSpec((D, D), lambda i: (0, 0)),
    pl.BlockSpec((1, D), lambda i: (0, 0)),
    pl.BlockSpec((D, D), lambda i: (0, 0)),
    pl.BlockSpec((1, D), lambda i: (0, 0)),
]

_mlp_split = pl.pallas_call(
    _mlp_split_body,
    grid=(GRID,),
    in_specs=_MLP_IN_SPECS,
    out_specs=pl.BlockSpec((2, RB, H), lambda i: (0, i, 0)),
    out_shape=jax.ShapeDtypeStruct((2, N, H), jnp.float32),
)


def _mlp_final_body(se_ref, h_ref, al_ref, ar_ref, w1_ref, b1_ref, w2_ref,
                    b2_ref, batch_ref, wl0_ref, bl0_ref, wlf_ref, blf_ref,
                    out_ref, xg_ref, sums_ref, cnt_ref):
    i = pl.program_id(0)
    z = _mlp(se_ref, h_ref, al_ref, ar_ref, w1_ref, b1_ref, w2_ref, b2_ref)
    out_ref[...] = z
    b = batch_ref[0, 0, :]
    oh = (b[:, None] == lax.broadcasted_iota(jnp.int32, (RB, NG), 1))
    oh = oh.astype(jnp.float32)
    dn = (((0,), (0,)), ((), ()))
    p = lax.dot_general(oh, z, dn, preferred_element_type=jnp.float32)
    cp = lax.dot_general(oh, jnp.ones((RB, 128), jnp.float32), dn,
                         preferred_element_type=jnp.float32)

    @pl.when(i == 0)
    def _():
        sums_ref[...] = p
        cnt_ref[...] = cp

    @pl.when(i > 0)
    def _():
        sums_ref[...] += p
        cnt_ref[...] += cp

    @pl.when(i == pl.num_programs(0) - 1)
    def _():
        cnt = jnp.maximum(cnt_ref[:, 0:1], 1.0)
        g = sums_ref[...] / cnt
        g = jnp.dot(g, wl0_ref[...], preferred_element_type=jnp.float32) + bl0_ref[...]
        g = _lrelu(g)
        g = jnp.dot(g, wlf_ref[...], preferred_element_type=jnp.float32) + blf_ref[...]
        xg_ref[...] = g


_mlp_final = pl.pallas_call(
    _mlp_final_body,
    grid=(GRID,),
    in_specs=_MLP_IN_SPECS + [
        pl.BlockSpec((1, 1, RB), lambda i: (i, 0, 0)),
        pl.BlockSpec((D, D), lambda i: (0, 0)),
        pl.BlockSpec((1, D), lambda i: (0, 0)),
        pl.BlockSpec((D, NCLS), lambda i: (0, 0)),
        pl.BlockSpec((1, NCLS), lambda i: (0, 0)),
    ],
    out_specs=(
        pl.BlockSpec((RB, D), lambda i: (i, 0)),
        pl.BlockSpec((NG, NCLS), lambda i: (0, 0)),
    ),
    out_shape=(
        jax.ShapeDtypeStruct((N, D), jnp.float32),
        jax.ShapeDtypeStruct((NG, NCLS), jnp.float32),
    ),
    scratch_shapes=[
        pltpu.VMEM((NG, D), jnp.float32),
        pltpu.VMEM((NG, 128), jnp.float32),
    ],
)


def kernel(x, edge_index, batch,
           eps0, W1_0, b1_0, g_mlp0, be_mlp0, W2_0, b2_0, g_out0, be_out0,
           eps1, W1_1, b1_1, g_mlp1, be_mlp1, W2_1, b2_1, g_out1, be_out1,
           eps2, W1_2, b1_2, g_mlp2, be_mlp2, W2_2, b2_2, g_out2, be_out2,
           Wl0, bl0, Wlf, blf):
    inv = (1.0 + BN_EPS) ** -0.5
    layers = []
    for eps, W1, b1, gm, bm, W2, b2, go, bo in (
            (eps0, W1_0, b1_0, g_mlp0, be_mlp0, W2_0, b2_0, g_out0, be_out0),
            (eps1, W1_1, b1_1, g_mlp1, be_mlp1, W2_1, b2_1, g_out1, be_out1),
            (eps2, W1_2, b1_2, g_mlp2, be_mlp2, W2_2, b2_2, g_out2, be_out2)):
        s1 = gm * inv
        s2 = go * inv
        layers.append((
            jnp.reshape(1.0 + eps, (1, 1)),
            W1 * s1[None, :], jnp.reshape(b1 * s1 + bm, (1, D)),
            W2 * s2[None, :], jnp.reshape(b2 * s2 + bo, (1, D)),
        ))

    pad = EPAD - E
    pad_src = jnp.arange(pad, dtype=jnp.int32) % N
    pad_dst = N + jnp.arange(pad, dtype=jnp.int32) % (NP - N)
    src_p = jnp.concatenate([edge_index[0], pad_src])
    dst_p = jnp.concatenate([edge_index[1], pad_dst])
    src3 = src_p.reshape(NT, NCH, CH)
    srcn3 = src3 + N
    dst3 = dst_p.reshape(NT, NCH, CH)
    batch3 = batch.reshape(GRID, 1, RB)

    h2 = jnp.stack([x[:, :H], x[:, H:]])
    for i, (se, w1, b1, w2, b2) in enumerate(layers):
        al, ar = _sc_agg(src3, srcn3, dst3, h2.reshape(2 * N, H))
        if i < 2:
            h2 = _mlp_split(se, h2, al, ar, w1, b1, w2, b2)
        else:
            out, xg = _mlp_final(se, h2, al, ar, w1, b1, w2, b2, batch3,
                                 Wl0, jnp.reshape(bl0, (1, D)),
                                 Wlf, jnp.reshape(blf, (1, NCLS)))
    return (xg, out)


# speedup vs baseline: 1.0188x; 1.0188x over previous
import functools

import jax
import jax.numpy as jnp
from jax import lax
from jax.experimental import pallas as pl
from jax.experimental.pallas import tpu as pltpu
from jax.experimental.pallas import tpu_sc as plsc

N = 10000
E = 160000
D = 256
NG = 16
NCLS = 10
BN_EPS = 1e-5

H = D // 2
NT = 16
CH = 128
NCH = 80
EPT = NCH * CH
EPAD = NT * EPT
G = 8
NGR = NCH // G
RPT = 640
NP = NT * RPT
RB = 2000
GRID = N // RB


def _sc_agg_body(src_hbm, srcn_hbm, dst_hbm, hlr_hbm, outl_hbm, outr_hbm,
                 sidx_a, didx_a, sidx_b, didx_b, rows0, rows1,
                 acc_sh, gsem0, gsem1, ssem0, ssem1):
    c = lax.axis_index("c")
    s = lax.axis_index("s")
    rows = (rows0, rows1)
    gsem = (gsem0, gsem1)
    ssem = (ssem0, ssem1)

    zvec = jnp.zeros((16,), jnp.float32)

    def zrow(i, carry):
        for k in range(8):
            rows0[i, pl.ds(k * 16, 16)] = zvec
        return carry

    lax.fori_loop(0, 64, zrow, 0)
    base = pl.multiple_of(s * RPT, RPT)
    for t in range(RPT // 64):
        pltpu.sync_copy(rows0.at[pl.ds(0, 64)], acc_sh.at[pl.ds(base + t * 64, 64)])
    plsc.subcore_barrier()

    def load_idx(sidx, didx, off):
        @pl.when(c == 0)
        def _():
            pltpu.sync_copy(src_hbm.at[s, pl.ds(off, G)], sidx)

        @pl.when(c == 1)
        def _():
            pltpu.sync_copy(srcn_hbm.at[s, pl.ds(off, G)], sidx)

        pltpu.sync_copy(dst_hbm.at[s, pl.ds(off, G)], didx)

    def gather(idx_ref, j, buf):
        return pltpu.async_copy(hlr_hbm.at[idx_ref.at[j]], rows[buf], gsem[buf])

    def pair(k, carry):
        load_idx(sidx_a, didx_a, pl.multiple_of(2 * k * G, G))
        pendg = gather(sidx_a, 0, 0)
        load_idx(sidx_b, didx_b, pl.multiple_of((2 * k + 1) * G, G))
        pends = [None, None]
        for t in range(2 * G):
            didx = didx_a if t < G else didx_b
            j = t % G
            cur = pendg
            if t + 1 < 2 * G:
                if pends[(t + 1) % 2] is not None:
                    pends[(t + 1) % 2].wait()
                nidx = sidx_a if t + 1 < G else sidx_b
                pendg = gather(nidx, (t + 1) % G, (t + 1) % 2)
            cur.wait()
            pends[t % 2] = pltpu.async_copy(
                rows[t % 2], acc_sh.at[didx.at[j]], ssem[t % 2], add=True)
        pends[0].wait()
        pends[1].wait()
        return carry

    lax.fori_loop(0, NGR // 2, pair, 0)
    plsc.subcore_barrier()

    base_w = pl.multiple_of(jnp.minimum(base, N - RPT), 8)

    @pl.when(c == 0)
    def _():
        pltpu.sync_copy(acc_sh.at[pl.ds(base_w, RPT)], outl_hbm.at[pl.ds(base_w, RPT)])

    @pl.when(c == 1)
    def _():
        pltpu.sync_copy(acc_sh.at[pl.ds(base_w, RPT)], outr_hbm.at[pl.ds(base_w, RPT)])


_sc_agg = pl.kernel(
    _sc_agg_body,
    out_type=(
        jax.ShapeDtypeStruct((N, H), jnp.float32),
        jax.ShapeDtypeStruct((N, H), jnp.float32),
    ),
    mesh=plsc.VectorSubcoreMesh(core_axis_name="c", subcore_axis_name="s"),
    scratch_types=(
        pltpu.VMEM((G, CH), jnp.int32),
        pltpu.VMEM((G, CH), jnp.int32),
        pltpu.VMEM((G, CH), jnp.int32),
        pltpu.VMEM((G, CH), jnp.int32),
        pltpu.VMEM((CH, H), jnp.float32),
        pltpu.VMEM((CH, H), jnp.float32),
        pltpu.VMEM_SHARED((NP, H), jnp.float32),
        pltpu.SemaphoreType.DMA,
        pltpu.SemaphoreType.DMA,
        pltpu.SemaphoreType.DMA,
        pltpu.SemaphoreType.DMA,
    ),
)


def _lrelu(x):
    return jnp.where(x > 0, x, 0.01 * x)


def _mlp(se_ref, h_ref, al_ref, ar_ref, w1_ref, b1_ref, w2_ref, b2_ref):
    h = jnp.concatenate([h_ref[0], h_ref[1]], axis=1)
    a = jnp.concatenate([al_ref[...], ar_ref[...]], axis=1)
    z = se_ref[0, 0] * h + a
    z = jnp.dot(z, w1_ref[...], preferred_element_type=jnp.float32) + b1_ref[...]
    z = _lrelu(z)
    z = jnp.dot(z, w2_ref[...], preferred_element_type=jnp.float32) + b2_ref[...]
    return _lrelu(z)


def _mlp_split_body(se_ref, h_ref, al_ref, ar_ref,
                    w1_ref, b1_ref, w2_ref, b2_ref, out_ref):
    z = _mlp(se_ref, h_ref, al_ref, ar_ref, w1_ref, b1_ref, w2_ref, b2_ref)
    out_ref[0] = z[:, :H]
    out_ref[1] = z[:, H:]


_MLP_IN_SPECS = [
    pl.BlockSpec(memory_space=pltpu.SMEM),
    pl.BlockSpec((2, RB, H), lambda i: (0, i, 0)),
    pl.BlockSpec((RB, H), lambda i: (i, 0)),
    pl.BlockSpec((RB, H), lambda i: (i, 0)),
    pl.BlockSpec((D, D), lambda i: (0, 0)),
    pl.BlockSpec((1, D), lambda i: (0, 0)),
    pl.BlockSpec((D, D), lambda i: (0, 0)),
    pl.BlockSpec((1, D), lambda i: (0, 0)),
]

_mlp_split = pl.pallas_call(
    _mlp_split_body,
    grid=(GRID,),
    in_specs=_MLP_IN_SPECS,
    out_specs=pl.BlockSpec((2, RB, H), lambda i: (0, i, 0)),
    out_shape=jax.ShapeDtypeStruct((2, N, H), jnp.float32),
)


def _mlp_final_body(se_ref, h_ref, al_ref, ar_ref, w1_ref, b1_ref, w2_ref,
                    b2_ref, batch_ref, wl0_ref, bl0_ref, wlf_ref, blf_ref,
                    out_ref, xg_ref, sums_ref, cnt_ref):
    i = pl.program_id(0)
    z = _mlp(se_ref, h_ref, al_ref, ar_ref, w1_ref, b1_ref, w2_ref, b2_ref)
    out_ref[...] = z
    b = batch_ref[0, 0, :]
    oh = (b[:, None] == lax.broadcasted_iota(jnp.int32, (RB, NG), 1))
    oh = oh.astype(jnp.float32)
    dn = (((0,), (0,)), ((), ()))
    p = lax.dot_general(oh, z, dn, preferred_element_type=jnp.float32)
    cp = lax.dot_general(oh, jnp.ones((RB, 128), jnp.float32), dn,
                         preferred_element_type=jnp.float32)

    @pl.when(i == 0)
    def _():
        sums_ref[...] = p
        cnt_ref[...] = cp

    @pl.when(i > 0)
    def _():
        sums_ref[...] += p
        cnt_ref[...] += cp

    @pl.when(i == pl.num_programs(0) - 1)
    def _():
        cnt = jnp.maximum(cnt_ref[:, 0:1], 1.0)
        g = sums_ref[...] / cnt
        g = jnp.dot(g, wl0_ref[...], preferred_element_type=jnp.float32) + bl0_ref[...]
        g = _lrelu(g)
        g = jnp.dot(g, wlf_ref[...], preferred_element_type=jnp.float32) + blf_ref[...]
        xg_ref[...] = g


_mlp_final = pl.pallas_call(
    _mlp_final_body,
    grid=(GRID,),
    in_specs=_MLP_IN_SPECS + [
        pl.BlockSpec((1, 1, RB), lambda i: (i, 0, 0)),
        pl.BlockSpec((D, D), lambda i: (0, 0)),
        pl.BlockSpec((1, D), lambda i: (0, 0)),
        pl.BlockSpec((D, NCLS), lambda i: (0, 0)),
        pl.BlockSpec((1, NCLS), lambda i: (0, 0)),
    ],
    out_specs=(
        pl.BlockSpec((RB, D), lambda i: (i, 0)),
        pl.BlockSpec((NG, NCLS), lambda i: (0, 0)),
    ),
    out_shape=(
        jax.ShapeDtypeStruct((N, D), jnp.float32),
        jax.ShapeDtypeStruct((NG, NCLS), jnp.float32),
    ),
    scratch_shapes=[
        pltpu.VMEM((NG, D), jnp.float32),
        pltpu.VMEM((NG, 128), jnp.float32),
    ],
)


def kernel(x, edge_index, batch,
           eps0, W1_0, b1_0, g_mlp0, be_mlp0, W2_0, b2_0, g_out0, be_out0,
           eps1, W1_1, b1_1, g_mlp1, be_mlp1, W2_1, b2_1, g_out1, be_out1,
           eps2, W1_2, b1_2, g_mlp2, be_mlp2, W2_2, b2_2, g_out2, be_out2,
           Wl0, bl0, Wlf, blf):
    inv = (1.0 + BN_EPS) ** -0.5
    layers = []
    for eps, W1, b1, gm, bm, W2, b2, go, bo in (
            (eps0, W1_0, b1_0, g_mlp0, be_mlp0, W2_0, b2_0, g_out0, be_out0),
            (eps1, W1_1, b1_1, g_mlp1, be_mlp1, W2_1, b2_1, g_out1, be_out1),
            (eps2, W1_2, b1_2, g_mlp2, be_mlp2, W2_2, b2_2, g_out2, be_out2)):
        s1 = gm * inv
        s2 = go * inv
        layers.append((
            jnp.reshape(1.0 + eps, (1, 1)),
            W1 * s1[None, :], jnp.reshape(b1 * s1 + bm, (1, D)),
            W2 * s2[None, :], jnp.reshape(b2 * s2 + bo, (1, D)),
        ))

    pad = EPAD - E
    pad_src = jnp.arange(pad, dtype=jnp.int32) % N
    pad_dst = N + jnp.arange(pad, dtype=jnp.int32) % (NP - N)
    src_p = jnp.concatenate([edge_index[0], pad_src])
    dst_p = jnp.concatenate([edge_index[1], pad_dst])
    src3 = src_p.reshape(NT, NCH, CH)
    srcn3 = src3 + N
    dst3 = dst_p.reshape(NT, NCH, CH)
    batch3 = batch.reshape(GRID, 1, RB)

    h2 = jnp.stack([x[:, :H], x[:, H:]])
    for i, (se, w1, b1, w2, b2) in enumerate(layers):
        al, ar = _sc_agg(src3, srcn3, dst3, h2.reshape(2 * N, H))
        if i < 2:
            h2 = _mlp_split(se, h2, al, ar, w1, b1, w2, b2)
        else:
            out, xg = _mlp_final(se, h2, al, ar, w1, b1, w2, b2, batch3,
                                 Wl0, jnp.reshape(bl0, (1, D)),
                                 Wlf, jnp.reshape(blf, (1, NCLS)))
    return (xg, out)


# speedup vs baseline: 1.0227x; 1.0038x over previous
import functools

import jax
import jax.numpy as jnp
from jax import lax
from jax.experimental import pallas as pl
from jax.experimental.pallas import tpu as pltpu
from jax.experimental.pallas import tpu_sc as plsc

N = 10000
E = 160000
D = 256
NG = 16
NCLS = 10
BN_EPS = 1e-5

H = D // 2
NT = 16
CH = 128
NCH = 80
EPT = NCH * CH
EPAD = NT * EPT
G = 8
NGR = NCH // G
RPT = 640
NP = NT * RPT
RB = 2000
GRID = N // RB


def _sc_agg_body(src_hbm, srcn_hbm, dst_hbm, hlr_hbm, outl_hbm, outr_hbm,
                 sidx_a, didx_a, sidx_b, didx_b, rows0, rows1,
                 acc_sh, gsem0, gsem1, ssem0, ssem1):
    c = lax.axis_index("c")
    s = lax.axis_index("s")
    rows = (rows0, rows1)
    gsem = (gsem0, gsem1)
    ssem = (ssem0, ssem1)

    zvec = jnp.zeros((16,), jnp.float32)

    def zrow(i, carry):
        for k in range(8):
            rows0[i, pl.ds(k * 16, 16)] = zvec
        return carry

    lax.fori_loop(0, 64, zrow, 0)
    base = pl.multiple_of(s * RPT, RPT)
    inits = [
        pltpu.async_copy(rows0.at[pl.ds(0, 64)],
                         acc_sh.at[pl.ds(base + t * 64, 64)], gsem0)
        for t in range(RPT // 64)
    ]
    for d in inits:
        d.wait()
    plsc.subcore_barrier()

    def load_idx(sidx, didx, off):
        @pl.when(c == 0)
        def _():
            pltpu.sync_copy(src_hbm.at[s, pl.ds(off, G)], sidx)

        @pl.when(c == 1)
        def _():
            pltpu.sync_copy(srcn_hbm.at[s, pl.ds(off, G)], sidx)

        pltpu.sync_copy(dst_hbm.at[s, pl.ds(off, G)], didx)

    def gather(idx_ref, j, buf):
        return pltpu.async_copy(hlr_hbm.at[idx_ref.at[j]], rows[buf], gsem[buf])

    def pair(k, carry):
        load_idx(sidx_a, didx_a, pl.multiple_of(2 * k * G, G))
        pendg = gather(sidx_a, 0, 0)
        load_idx(sidx_b, didx_b, pl.multiple_of((2 * k + 1) * G, G))
        pends = [None, None]
        for t in range(2 * G):
            didx = didx_a if t < G else didx_b
            j = t % G
            cur = pendg
            if t + 1 < 2 * G:
                if pends[(t + 1) % 2] is not None:
                    pends[(t + 1) % 2].wait()
                nidx = sidx_a if t + 1 < G else sidx_b
                pendg = gather(nidx, (t + 1) % G, (t + 1) % 2)
            cur.wait()
            pends[t % 2] = pltpu.async_copy(
                rows[t % 2], acc_sh.at[didx.at[j]], ssem[t % 2], add=True)
        pends[0].wait()
        pends[1].wait()
        return carry

    lax.fori_loop(0, NGR // 2, pair, 0)
    plsc.subcore_barrier()

    base_w = pl.multiple_of(jnp.minimum(base, N - RPT), 8)

    @pl.when(c == 0)
    def _():
        pltpu.sync_copy(acc_sh.at[pl.ds(base_w, RPT)], outl_hbm.at[pl.ds(base_w, RPT)])

    @pl.when(c == 1)
    def _():
        pltpu.sync_copy(acc_sh.at[pl.ds(base_w, RPT)], outr_hbm.at[pl.ds(base_w, RPT)])


_sc_agg = pl.kernel(
    _sc_agg_body,
    out_type=(
        jax.ShapeDtypeStruct((N, H), jnp.float32),
        jax.ShapeDtypeStruct((N, H), jnp.float32),
    ),
    mesh=plsc.VectorSubcoreMesh(core_axis_name="c", subcore_axis_name="s"),
    scratch_types=(
        pltpu.VMEM((G, CH), jnp.int32),
        pltpu.VMEM((G, CH), jnp.int32),
        pltpu.VMEM((G, CH), jnp.int32),
        pltpu.VMEM((G, CH), jnp.int32),
        pltpu.VMEM((CH, H), jnp.float32),
        pltpu.VMEM((CH, H), jnp.float32),
        pltpu.VMEM_SHARED((NP, H), jnp.float32),
        pltpu.SemaphoreType.DMA,
        pltpu.SemaphoreType.DMA,
        pltpu.SemaphoreType.DMA,
        pltpu.SemaphoreType.DMA,
    ),
)


def _lrelu(x):
    return jnp.where(x > 0, x, 0.01 * x)


def _mlp(se_ref, h_ref, al_ref, ar_ref, w1_ref, b1_ref, w2_ref, b2_ref):
    h = jnp.concatenate([h_ref[0], h_ref[1]], axis=1)
    a = jnp.concatenate([al_ref[...], ar_ref[...]], axis=1)
    z = se_ref[0, 0] * h + a
    z = jnp.dot(z, w1_ref[...], preferred_element_type=jnp.float32) + b1_ref[...]
    z = _lrelu(z)
    z = jnp.dot(z, w2_ref[...], preferred_element_type=jnp.float32) + b2_ref[...]
    return _lrelu(z)


def _mlp_split_body(se_ref, h_ref, al_ref, ar_ref,
                    w1_ref, b1_ref, w2_ref, b2_ref, out_ref):
    z = _mlp(se_ref, h_ref, al_ref, ar_ref, w1_ref, b1_ref, w2_ref, b2_ref)
    out_ref[0] = z[:, :H]
    out_ref[1] = z[:, H:]


_MLP_IN_SPECS = [
    pl.BlockSpec(memory_space=pltpu.SMEM),
    pl.BlockSpec((2, RB, H), lambda i: (0, i, 0)),
    pl.BlockSpec((RB, H), lambda i: (i, 0)),
    pl.BlockSpec((RB, H), lambda i: (i, 0)),
    pl.BlockSpec((D, D), lambda i: (0, 0)),
    pl.BlockSpec((1, D), lambda i: (0, 0)),
    pl.BlockSpec((D, D), lambda i: (0, 0)),
    pl.BlockSpec((1, D), lambda i: (0, 0)),
]

_mlp_split = pl.pallas_call(
    _mlp_split_body,
    grid=(GRID,),
    in_specs=_MLP_IN_SPECS,
    out_specs=pl.BlockSpec((2, RB, H), lambda i: (0, i, 0)),
    out_shape=jax.ShapeDtypeStruct((2, N, H), jnp.float32),
)


def _mlp_final_body(se_ref, h_ref, al_ref, ar_ref, w1_ref, b1_ref, w2_ref,
                    b2_ref, batch_ref, wl0_ref, bl0_ref, wlf_ref, blf_ref,
                    out_ref, xg_ref, sums_ref, cnt_ref):
    i = pl.program_id(0)
    z = _mlp(se_ref, h_ref, al_ref, ar_ref, w1_ref, b1_ref, w2_ref, b2_ref)
    out_ref[...] = z
    b = batch_ref[0, 0, :]
    oh = (b[:, None] == lax.broadcasted_iota(jnp.int32, (RB, NG), 1))
    oh = oh.astype(jnp.float32)
    dn = (((0,), (0,)), ((), ()))
    p = lax.dot_general(oh, z, dn, preferred_element_type=jnp.float32)
    cp = lax.dot_general(oh, jnp.ones((RB, 128), jnp.float32), dn,
                         preferred_element_type=jnp.float32)

    @pl.when(i == 0)
    def _():
        sums_ref[...] = p
        cnt_ref[...] = cp

    @pl.when(i > 0)
    def _():
        sums_ref[...] += p
        cnt_ref[...] += cp

    @pl.when(i == pl.num_programs(0) - 1)
    def _():
        cnt = jnp.maximum(cnt_ref[:, 0:1], 1.0)
        g = sums_ref[...] / cnt
        g = jnp.dot(g, wl0_ref[...], preferred_element_type=jnp.float32) + bl0_ref[...]
        g = _lrelu(g)
        g = jnp.dot(g, wlf_ref[...], preferred_element_type=jnp.float32) + blf_ref[...]
        xg_ref[...] = g


_mlp_final = pl.pallas_call(
    _mlp_final_body,
    grid=(GRID,),
    in_specs=_MLP_IN_SPECS + [
        pl.BlockSpec((1, 1, RB), lambda i: (i, 0, 0)),
        pl.BlockSpec((D, D), lambda i: (0, 0)),
        pl.BlockSpec((1, D), lambda i: (0, 0)),
        pl.BlockSpec((D, NCLS), lambda i: (0, 0)),
        pl.BlockSpec((1, NCLS), lambda i: (0, 0)),
    ],
    out_specs=(
        pl.BlockSpec((RB, D), lambda i: (i, 0)),
        pl.BlockSpec((NG, NCLS), lambda i: (0, 0)),
    ),
    out_shape=(
        jax.ShapeDtypeStruct((N, D), jnp.float32),
        jax.ShapeDtypeStruct((NG, NCLS), jnp.float32),
    ),
    scratch_shapes=[
        pltpu.VMEM((NG, D), jnp.float32),
        pltpu.VMEM((NG, 128), jnp.float32),
    ],
)


def kernel(x, edge_index, batch,
           eps0, W1_0, b1_0, g_mlp0, be_mlp0, W2_0, b2_0, g_out0, be_out0,
           eps1, W1_1, b1_1, g_mlp1, be_mlp1, W2_1, b2_1, g_out1, be_out1,
           eps2, W1_2, b1_2, g_mlp2, be_mlp2, W2_2, b2_2, g_out2, be_out2,
           Wl0, bl0, Wlf, blf):
    inv = (1.0 + BN_EPS) ** -0.5
    layers = []
    for eps, W1, b1, gm, bm, W2, b2, go, bo in (
            (eps0, W1_0, b1_0, g_mlp0, be_mlp0, W2_0, b2_0, g_out0, be_out0),
            (eps1, W1_1, b1_1, g_mlp1, be_mlp1, W2_1, b2_1, g_out1, be_out1),
            (eps2, W1_2, b1_2, g_mlp2, be_mlp2, W2_2, b2_2, g_out2, be_out2)):
        s1 = gm * inv
        s2 = go * inv
        layers.append((
            jnp.reshape(1.0 + eps, (1, 1)),
            W1 * s1[None, :], jnp.reshape(b1 * s1 + bm, (1, D)),
            W2 * s2[None, :], jnp.reshape(b2 * s2 + bo, (1, D)),
        ))

    pad = EPAD - E
    pad_src = jnp.arange(pad, dtype=jnp.int32) % N
    pad_dst = N + jnp.arange(pad, dtype=jnp.int32) % (NP - N)
    src_p = jnp.concatenate([edge_index[0], pad_src])
    dst_p = jnp.concatenate([edge_index[1], pad_dst])
    src3 = src_p.reshape(NT, NCH, CH)
    srcn3 = src3 + N
    dst3 = dst_p.reshape(NT, NCH, CH)
    batch3 = batch.reshape(GRID, 1, RB)

    h2 = jnp.stack([x[:, :H], x[:, H:]])
    for i, (se, w1, b1, w2, b2) in enumerate(layers):
        al, ar = _sc_agg(src3, srcn3, dst3, h2.reshape(2 * N, H))
        if i < 2:
            h2 = _mlp_split(se, h2, al, ar, w1, b1, w2, b2)
        else:
            out, xg = _mlp_final(se, h2, al, ar, w1, b1, w2, b2, batch3,
                                 Wl0, jnp.reshape(bl0, (1, D)),
                                 Wlf, jnp.reshape(blf, (1, NCLS)))
    return (xg, out)


# speedup vs baseline: 1.0235x; 1.0008x over previous
import functools

import jax
import jax.numpy as jnp
from jax import lax
from jax.experimental import pallas as pl
from jax.experimental.pallas import tpu as pltpu
from jax.experimental.pallas import tpu_sc as plsc

N = 10000
E = 160000
D = 256
NG = 16
NCLS = 10
BN_EPS = 1e-5

H = D // 2
NT = 16
CH = 128
NCH = 80
EPT = NCH * CH
EPAD = NT * EPT
G = 8
NGR = NCH // G
RPT = 640
NP = NT * RPT
RB = 2000
GRID = N // RB


def _sc_agg_body(cmb0_hbm, cmb1_hbm, hlr_hbm, outl_hbm, outr_hbm,
                 cidx, rows0, rows1,
                 acc_sh, gsem0, gsem1, ssem0, ssem1):
    c = lax.axis_index("c")
    s = lax.axis_index("s")
    rows = (rows0, rows1)
    gsem = (gsem0, gsem1)
    ssem = (ssem0, ssem1)

    zvec = jnp.zeros((16,), jnp.float32)

    def zrow(i, carry):
        for k in range(8):
            rows0[i, pl.ds(k * 16, 16)] = zvec
        return carry

    lax.fori_loop(0, 64, zrow, 0)
    base = pl.multiple_of(s * RPT, RPT)
    inits = [
        pltpu.async_copy(rows0.at[pl.ds(0, 64)],
                         acc_sh.at[pl.ds(base + t * 64, 64)], gsem0)
        for t in range(RPT // 64)
    ]
    for d in inits:
        d.wait()
    plsc.subcore_barrier()

    def gather(j, buf):
        return pltpu.async_copy(hlr_hbm.at[cidx.at[j]], rows[buf], gsem[buf])

    def pair(k, carry):
        @pl.when(c == 0)
        def _():
            pltpu.sync_copy(cmb0_hbm.at[s, k], cidx)

        @pl.when(c == 1)
        def _():
            pltpu.sync_copy(cmb1_hbm.at[s, k], cidx)

        pendg = gather(0, 0)
        pends = [None, None]
        for t in range(2 * G):
            jd = G + t if t < G else 2 * G + t
            cur = pendg
            if t + 1 < 2 * G:
                jg = t + 1 if t + 1 < G else G + t + 1
                if pends[(t + 1) % 2] is not None:
                    pends[(t + 1) % 2].wait()
                pendg = gather(jg, (t + 1) % 2)
            cur.wait()
            pends[t % 2] = pltpu.async_copy(
                rows[t % 2], acc_sh.at[cidx.at[jd]], ssem[t % 2], add=True)
        pends[0].wait()
        pends[1].wait()
        return carry

    lax.fori_loop(0, NGR // 2, pair, 0)
    plsc.subcore_barrier()

    base_w = pl.multiple_of(jnp.minimum(base, N - RPT), 8)

    @pl.when(c == 0)
    def _():
        pltpu.sync_copy(acc_sh.at[pl.ds(base_w, RPT)], outl_hbm.at[pl.ds(base_w, RPT)])

    @pl.when(c == 1)
    def _():
        pltpu.sync_copy(acc_sh.at[pl.ds(base_w, RPT)], outr_hbm.at[pl.ds(base_w, RPT)])


_sc_agg = pl.kernel(
    _sc_agg_body,
    out_type=(
        jax.ShapeDtypeStruct((N, H), jnp.float32),
        jax.ShapeDtypeStruct((N, H), jnp.float32),
    ),
    mesh=plsc.VectorSubcoreMesh(core_axis_name="c", subcore_axis_name="s"),
    scratch_types=(
        pltpu.VMEM((4 * G, CH), jnp.int32),
        pltpu.VMEM((CH, H), jnp.float32),
        pltpu.VMEM((CH, H), jnp.float32),
        pltpu.VMEM_SHARED((NP, H), jnp.float32),
        pltpu.SemaphoreType.DMA,
        pltpu.SemaphoreType.DMA,
        pltpu.SemaphoreType.DMA,
        pltpu.SemaphoreType.DMA,
    ),
)


def _lrelu(x):
    return jnp.where(x > 0, x, 0.01 * x)


def _mlp(se_ref, h_ref, al_ref, ar_ref, w1_ref, b1_ref, w2_ref, b2_ref):
    h = jnp.concatenate([h_ref[0], h_ref[1]], axis=1)
    a = jnp.concatenate([al_ref[...], ar_ref[...]], axis=1)
    z = se_ref[0, 0] * h + a
    z = jnp.dot(z, w1_ref[...], preferred_element_type=jnp.float32) + b1_ref[...]
    z = _lrelu(z)
    z = jnp.dot(z, w2_ref[...], preferred_element_type=jnp.float32) + b2_ref[...]
    return _lrelu(z)


def _mlp_split_body(se_ref, h_ref, al_ref, ar_ref,
                    w1_ref, b1_ref, w2_ref, b2_ref, out_ref):
    z = _mlp(se_ref, h_ref, al_ref, ar_ref, w1_ref, b1_ref, w2_ref, b2_ref)
    out_ref[0] = z[:, :H]
    out_ref[1] = z[:, H:]


_MLP_IN_SPECS = [
    pl.BlockSpec(memory_space=pltpu.SMEM),
    pl.BlockSpec((2, RB, H), lambda i: (0, i, 0)),
    pl.BlockSpec((RB, H), lambda i: (i, 0)),
    pl.BlockSpec((RB, H), lambda i: (i, 0)),
    pl.BlockSpec((D, D), lambda i: (0, 0)),
    pl.BlockSpec((1, D), lambda i: (0, 0)),
    pl.BlockSpec((D, D), lambda i: (0, 0)),
    pl.BlockSpec((1, D), lambda i: (0, 0)),
]

_mlp_split = pl.pallas_call(
    _mlp_split_body,
    grid=(GRID,),
    in_specs=_MLP_IN_SPECS,
    out_specs=pl.BlockSpec((2, RB, H), lambda i: (0, i, 0)),
    out_shape=jax.ShapeDtypeStruct((2, N, H), jnp.float32),
)


def _mlp_final_body(se_ref, h_ref, al_ref, ar_ref, w1_ref, b1_ref, w2_ref,
                    b2_ref, batch_ref, wl0_ref, bl0_ref, wlf_ref, blf_ref,
                    out_ref, xg_ref, sums_ref, cnt_ref):
    i = pl.program_id(0)
    z = _mlp(se_ref, h_ref, al_ref, ar_ref, w1_ref, b1_ref, w2_ref, b2_ref)
    out_ref[...] = z
    b = batch_ref[0, 0, :]
    oh = (b[:, None] == lax.broadcasted_iota(jnp.int32, (RB, NG), 1))
    oh = oh.astype(jnp.float32)
    dn = (((0,), (0,)), ((), ()))
    p = lax.dot_general(oh, z, dn, preferred_element_type=jnp.float32)
    cp = lax.dot_general(oh, jnp.ones((RB, 128), jnp.float32), dn,
                         preferred_element_type=jnp.float32)

    @pl.when(i == 0)
    def _():
        sums_ref[...] = p
        cnt_ref[...] = cp

    @pl.when(i > 0)
    def _():
        sums_ref[...] += p
        cnt_ref[...] += cp

    @pl.when(i == pl.num_programs(0) - 1)
    def _():
        cnt = jnp.maximum(cnt_ref[:, 0:1], 1.0)
        g = sums_ref[...] / cnt
        g = jnp.dot(g, wl0_ref[...], preferred_element_type=jnp.float32) + bl0_ref[...]
        g = _lrelu(g)
        g = jnp.dot(g, wlf_ref[...], preferred_element_type=jnp.float32) + blf_ref[...]
        xg_ref[...] = g


_mlp_final = pl.pallas_call(
    _mlp_final_body,
    grid=(GRID,),
    in_specs=_MLP_IN_SPECS + [
        pl.BlockSpec((1, 1, RB), lambda i: (i, 0, 0)),
        pl.BlockSpec((D, D), lambda i: (0, 0)),
        pl.BlockSpec((1, D), lambda i: (0, 0)),
        pl.BlockSpec((D, NCLS), lambda i: (0, 0)),
        pl.BlockSpec((1, NCLS), lambda i: (0, 0)),
    ],
    out_specs=(
        pl.BlockSpec((RB, D), lambda i: (i, 0)),
        pl.BlockSpec((NG, NCLS), lambda i: (0, 0)),
    ),
    out_shape=(
        jax.ShapeDtypeStruct((N, D), jnp.float32),
        jax.ShapeDtypeStruct((NG, NCLS), jnp.float32),
    ),
    scratch_shapes=[
        pltpu.VMEM((NG, D), jnp.float32),
        pltpu.VMEM((NG, 128), jnp.float32),
    ],
)


def kernel(x, edge_index, batch,
           eps0, W1_0, b1_0, g_mlp0, be_mlp0, W2_0, b2_0, g_out0, be_out0,
           eps1, W1_1, b1_1, g_mlp1, be_mlp1, W2_1, b2_1, g_out1, be_out1,
           eps2, W1_2, b1_2, g_mlp2, be_mlp2, W2_2, b2_2, g_out2, be_out2,
           Wl0, bl0, Wlf, blf):
    inv = (1.0 + BN_EPS) ** -0.5
    layers = []
    for eps, W1, b1, gm, bm, W2, b2, go, bo in (
            (eps0, W1_0, b1_0, g_mlp0, be_mlp0, W2_0, b2_0, g_out0, be_out0),
            (eps1, W1_1, b1_1, g_mlp1, be_mlp1, W2_1, b2_1, g_out1, be_out1),
            (eps2, W1_2, b1_2, g_mlp2, be_mlp2, W2_2, b2_2, g_out2, be_out2)):
        s1 = gm * inv
        s2 = go * inv
        layers.append((
            jnp.reshape(1.0 + eps, (1, 1)),
            W1 * s1[None, :], jnp.reshape(b1 * s1 + bm, (1, D)),
            W2 * s2[None, :], jnp.reshape(b2 * s2 + bo, (1, D)),
        ))

    pad = EPAD - E
    pad_src = jnp.arange(pad, dtype=jnp.int32) % N
    pad_dst = N + jnp.arange(pad, dtype=jnp.int32) % (NP - N)
    src_p = jnp.concatenate([edge_index[0], pad_src])
    dst_p = jnp.concatenate([edge_index[1], pad_dst])
    sp = src_p.reshape(NT, NGR // 2, 2, G, CH)
    dp = dst_p.reshape(NT, NGR // 2, 2, G, CH)
    cmb0 = jnp.concatenate([sp[:, :, 0], dp[:, :, 0], sp[:, :, 1], dp[:, :, 1]],
                           axis=2)
    cmb1 = jnp.concatenate([sp[:, :, 0] + N, dp[:, :, 0], sp[:, :, 1] + N,
                            dp[:, :, 1]], axis=2)
    batch3 = batch.reshape(GRID, 1, RB)

    h2 = jnp.stack([x[:, :H], x[:, H:]])
    for i, (se, w1, b1, w2, b2) in enumerate(layers):
        al, ar = _sc_agg(cmb0, cmb1, h2.reshape(2 * N, H))
        if i < 2:
            h2 = _mlp_split(se, h2, al, ar, w1, b1, w2, b2)
        else:
            out, xg = _mlp_final(se, h2, al, ar, w1, b1, w2, b2, batch3,
                                 Wl0, jnp.reshape(bl0, (1, D)),
                                 Wlf, jnp.reshape(blf, (1, NCLS)))
    return (xg, out)


# speedup vs baseline: 1.0278x; 1.0042x over previous
import jax
import jax.numpy as jnp
from jax import lax
from jax.experimental import pallas as pl
from jax.experimental.pallas import tpu as pltpu
from jax.experimental.pallas import tpu_sc as plsc

N = 10000
E = 160000
D = 256
NG = 16
NCLS = 10
BN_EPS = 1e-5

H = D // 2
NT = 16
CH = 128
NCH = 80
EPT = NCH * CH
EPAD = NT * EPT
G = 8
NGR = NCH // G
RPT = 640
NP = NT * RPT
RB = 2000
GRID = N // RB


def _sc_agg_body(cmb0_hbm, cmb1_hbm, hlr_hbm, outl_hbm, outr_hbm,
                 cidx, rows0, rows1,
                 acc_sh, gsem0, gsem1, ssem0, ssem1):
    c = lax.axis_index("c")
    s = lax.axis_index("s")
    rows = (rows0, rows1)
    gsem = (gsem0, gsem1)
    ssem = (ssem0, ssem1)

    zvec = jnp.zeros((16,), jnp.float32)

    def zrow(i, carry):
        for k in range(8):
            rows0[i, pl.ds(k * 16, 16)] = zvec
        return carry

    lax.fori_loop(0, 64, zrow, 0)
    base = pl.multiple_of(s * RPT, RPT)
    inits = [
        pltpu.async_copy(rows0.at[pl.ds(0, 64)],
                         acc_sh.at[pl.ds(base + t * 64, 64)], gsem0)
        for t in range(RPT // 64)
    ]
    for d in inits:
        d.wait()
    plsc.subcore_barrier()

    def gather(j, buf):
        return pltpu.async_copy(hlr_hbm.at[cidx.at[j]], rows[buf], gsem[buf])

    def pair(k, carry):
        @pl.when(c == 0)
        def _():
            pltpu.sync_copy(cmb0_hbm.at[s, k], cidx)

        @pl.when(c == 1)
        def _():
            pltpu.sync_copy(cmb1_hbm.at[s, k], cidx)

        pendg = gather(0, 0)
        pends = [None, None]
        for t in range(2 * G):
            jd = G + t if t < G else 2 * G + t
            cur = pendg
            if t + 1 < 2 * G:
                jg = t + 1 if t + 1 < G else G + t + 1
                if pends[(t + 1) % 2] is not None:
                    pends[(t + 1) % 2].wait()
                pendg = gather(jg, (t + 1) % 2)
            cur.wait()
            pends[t % 2] = pltpu.async_copy(
                rows[t % 2], acc_sh.at[cidx.at[jd]], ssem[t % 2], add=True)
        pends[0].wait()
        pends[1].wait()
        return carry

    lax.fori_loop(0, NGR // 2, pair, 0)
    plsc.subcore_barrier()

    base_w = pl.multiple_of(jnp.minimum(base, N - RPT), 8)

    @pl.when(c == 0)
    def _():
        pltpu.sync_copy(acc_sh.at[pl.ds(base_w, RPT)], outl_hbm.at[pl.ds(base_w, RPT)])

    @pl.when(c == 1)
    def _():
        pltpu.sync_copy(acc_sh.at[pl.ds(base_w, RPT)], outr_hbm.at[pl.ds(base_w, RPT)])


_sc_agg = pl.kernel(
    _sc_agg_body,
    out_type=(
        jax.ShapeDtypeStruct((N, H), jnp.float32),
        jax.ShapeDtypeStruct((N, H), jnp.float32),
    ),
    mesh=plsc.VectorSubcoreMesh(core_axis_name="c", subcore_axis_name="s"),
    scratch_types=(
        pltpu.VMEM((4 * G, CH), jnp.int32),
        pltpu.VMEM((CH, H), jnp.float32),
        pltpu.VMEM((CH, H), jnp.float32),
        pltpu.VMEM_SHARED((NP, H), jnp.float32),
        pltpu.SemaphoreType.DMA,
        pltpu.SemaphoreType.DMA,
        pltpu.SemaphoreType.DMA,
        pltpu.SemaphoreType.DMA,
    ),
)


def _lrelu(x):
    return jnp.where(x > 0, x, 0.01 * x)


def _mlp(se_ref, h_ref, al_ref, ar_ref, w1_ref, b1_ref, w2_ref, b2_ref):
    h = jnp.concatenate([h_ref[0], h_ref[1]], axis=1)
    a = jnp.concatenate([al_ref[...], ar_ref[...]], axis=1)
    z = se_ref[0, 0] * h + a
    z = jnp.dot(z, w1_ref[...], preferred_element_type=jnp.float32) + b1_ref[...]
    z = _lrelu(z)
    z = jnp.dot(z, w2_ref[...], preferred_element_type=jnp.float32) + b2_ref[...]
    return _lrelu(z)


def _mlp_split_body(se_ref, h_ref, al_ref, ar_ref,
                    w1_ref, b1_ref, w2_ref, b2_ref, out_ref):
    z = _mlp(se_ref, h_ref, al_ref, ar_ref, w1_ref, b1_ref, w2_ref, b2_ref)
    out_ref[0] = z[:, :H]
    out_ref[1] = z[:, H:]


_MLP_IN_SPECS = [
    pl.BlockSpec(memory_space=pltpu.SMEM),
    pl.BlockSpec((2, RB, H), lambda i: (0, i, 0)),
    pl.BlockSpec((RB, H), lambda i: (i, 0)),
    pl.BlockSpec((RB, H), lambda i: (i, 0)),
    pl.BlockSpec((D, D), lambda i: (0, 0)),
    pl.BlockSpec((1, D), lambda i: (0, 0)),
    pl.BlockSpec((D, D), lambda i: (0, 0)),
    pl.BlockSpec((1, D), lambda i: (0, 0)),
]

_mlp_split = pl.pallas_call(
    _mlp_split_body,
    grid=(GRID,),
    in_specs=_MLP_IN_SPECS,
    out_specs=pl.BlockSpec((2, RB, H), lambda i: (0, i, 0)),
    out_shape=jax.ShapeDtypeStruct((2, N, H), jnp.float32),
)


def _mlp_final_body(se_ref, h_ref, al_ref, ar_ref, w1_ref, b1_ref, w2_ref,
                    b2_ref, batch_ref, wl0_ref, bl0_ref, wlf_ref, blf_ref,
                    out_ref, xg_ref, sums_ref, cnt_ref):
    i = pl.program_id(0)
    z = _mlp(se_ref, h_ref, al_ref, ar_ref, w1_ref, b1_ref, w2_ref, b2_ref)
    out_ref[...] = z
    b = batch_ref[0, 0, :]
    oh = (b[:, None] == lax.broadcasted_iota(jnp.int32, (RB, NG), 1))
    oh = oh.astype(jnp.float32)
    dn = (((0,), (0,)), ((), ()))
    p = lax.dot_general(oh, z, dn, preferred_element_type=jnp.float32)
    cp = lax.dot_general(oh, jnp.ones((RB, 128), jnp.float32), dn,
                         preferred_element_type=jnp.float32)

    @pl.when(i == 0)
    def _():
        sums_ref[...] = p
        cnt_ref[...] = cp

    @pl.when(i > 0)
    def _():
        sums_ref[...] += p
        cnt_ref[...] += cp

    @pl.when(i == pl.num_programs(0) - 1)
    def _():
        cnt = jnp.maximum(cnt_ref[:, 0:1], 1.0)
        g = sums_ref[...] / cnt
        g = jnp.dot(g, wl0_ref[...], preferred_element_type=jnp.float32) + bl0_ref[...]
        g = _lrelu(g)
        g = jnp.dot(g, wlf_ref[...], preferred_element_type=jnp.float32) + blf_ref[...]
        xg_ref[...] = g


_mlp_final = pl.pallas_call(
    _mlp_final_body,
    grid=(GRID,),
    in_specs=_MLP_IN_SPECS + [
        pl.BlockSpec((1, 1, RB), lambda i: (i, 0, 0)),
        pl.BlockSpec((D, D), lambda i: (0, 0)),
        pl.BlockSpec((1, D), lambda i: (0, 0)),
        pl.BlockSpec((D, NCLS), lambda i: (0, 0)),
        pl.BlockSpec((1, NCLS), lambda i: (0, 0)),
    ],
    out_specs=(
        pl.BlockSpec((RB, D), lambda i: (i, 0)),
        pl.BlockSpec((NG, NCLS), lambda i: (0, 0)),
    ),
    out_shape=(
        jax.ShapeDtypeStruct((N, D), jnp.float32),
        jax.ShapeDtypeStruct((NG, NCLS), jnp.float32),
    ),
    scratch_shapes=[
        pltpu.VMEM((NG, D), jnp.float32),
        pltpu.VMEM((NG, 128), jnp.float32),
    ],
)


def kernel(x, edge_index, batch,
           eps0, W1_0, b1_0, g_mlp0, be_mlp0, W2_0, b2_0, g_out0, be_out0,
           eps1, W1_1, b1_1, g_mlp1, be_mlp1, W2_1, b2_1, g_out1, be_out1,
           eps2, W1_2, b1_2, g_mlp2, be_mlp2, W2_2, b2_2, g_out2, be_out2,
           Wl0, bl0, Wlf, blf):
    inv = (1.0 + BN_EPS) ** -0.5
    layers = []
    for eps, W1, b1, gm, bm, W2, b2, go, bo in (
            (eps0, W1_0, b1_0, g_mlp0, be_mlp0, W2_0, b2_0, g_out0, be_out0),
            (eps1, W1_1, b1_1, g_mlp1, be_mlp1, W2_1, b2_1, g_out1, be_out1),
            (eps2, W1_2, b1_2, g_mlp2, be_mlp2, W2_2, b2_2, g_out2, be_out2)):
        s1 = gm * inv
        s2 = go * inv
        layers.append((
            jnp.reshape(1.0 + eps, (1, 1)),
            W1 * s1[None, :], jnp.reshape(b1 * s1 + bm, (1, D)),
            W2 * s2[None, :], jnp.reshape(b2 * s2 + bo, (1, D)),
        ))

    pad = EPAD - E
    pad_src = jnp.arange(pad, dtype=jnp.int32) % N
    pad_dst = N + jnp.arange(pad, dtype=jnp.int32) % (NP - N)
    src_p = jnp.concatenate([edge_index[0], pad_src])
    dst_p = jnp.concatenate([edge_index[1], pad_dst])
    sp = src_p.reshape(NT, NGR // 2, 2, G, CH)
    dp = dst_p.reshape(NT, NGR // 2, 2, G, CH)
    cmb0 = jnp.concatenate([sp[:, :, 0], dp[:, :, 0], sp[:, :, 1], dp[:, :, 1]],
                           axis=2)
    cmb1 = jnp.concatenate([sp[:, :, 0] + N, dp[:, :, 0], sp[:, :, 1] + N,
                            dp[:, :, 1]], axis=2)
    batch3 = batch.reshape(GRID, 1, RB)

    h2 = jnp.stack([x[:, :H], x[:, H:]])
    for i, (se, w1, b1, w2, b2) in enumerate(layers):
        al, ar = _sc_agg(cmb0, cmb1, h2.reshape(2 * N, H))
        if i < 2:
            h2 = _mlp_split(se, h2, al, ar, w1, b1, w2, b2)
        else:
            out, xg = _mlp_final(se, h2, al, ar, w1, b1, w2, b2, batch3,
                                 Wl0, jnp.reshape(bl0, (1, D)),
                                 Wlf, jnp.reshape(blf, (1, NCLS)))
    return (xg, out)
